# Initial kernel scaffold; baseline (speedup 1.0000x reference)
import functools

import jax
import jax.numpy as jnp
from jax.experimental import pallas as pl
from jax.experimental.pallas import tpu as pltpu

NUM_EXPERTS = 8
TOP_K = 2
HIDDEN = 768
INTER = 2048


def _router_body(x_ref, wr_ref, comb_ref):
    x = x_ref[...]
    wr = wr_ref[...]
    logits = jax.lax.dot_general(
        x, wr, (((1,), (0,)), ((), ())), preferred_element_type=jnp.float32
    )
    aff = jax.nn.sigmoid(logits)
    t = aff.shape[0]
    iota = jax.lax.broadcasted_iota(jnp.int32, (t, NUM_EXPERTS), 1)
    m1 = jnp.max(aff, axis=-1, keepdims=True)
    im1 = jnp.min(jnp.where(aff == m1, iota, NUM_EXPERTS), axis=-1, keepdims=True)
    aff2 = jnp.where(iota == im1, -jnp.inf, aff)
    m2 = jnp.max(aff2, axis=-1, keepdims=True)
    im2 = jnp.min(jnp.where(aff2 == m2, iota, NUM_EXPERTS), axis=-1, keepdims=True)
    s = m1 + m2
    comb_ref[...] = (
        jnp.where(iota == im1, m1 / s, 0.0) + jnp.where(iota == im2, m2 / s, 0.0)
    ).astype(jnp.float32)


def _ffn_body(x_ref, wg_ref, wu_ref, wd_ref, comb_ref, out_ref, acc_ref):
    e = pl.program_id(0)
    j = pl.program_id(1)
    nj = pl.num_programs(1)

    x = x_ref[...]
    g = jax.lax.dot_general(
        x, wg_ref[0], (((1,), (0,)), ((), ())), preferred_element_type=jnp.float32
    )
    u = jax.lax.dot_general(
        x, wu_ref[0], (((1,), (0,)), ((), ())), preferred_element_type=jnp.float32
    )
    h = (g * jax.nn.sigmoid(g)) * u
    y = jax.lax.dot_general(
        h, wd_ref[0], (((1,), (0,)), ((), ())), preferred_element_type=jnp.float32
    )
    y = y * comb_ref[:, e][:, None]

    tm = y.shape[0]
    row = j * tm

    @pl.when(e == 0)
    def _init():
        acc_ref[pl.ds(row, tm), :] = y

    @pl.when(e > 0)
    def _acc():
        acc_ref[pl.ds(row, tm), :] = acc_ref[pl.ds(row, tm), :] + y

    @pl.when((e == pl.num_programs(0) - 1) & (j == nj - 1))
    def _fin():
        out_ref[...] = acc_ref[...]


@jax.jit
def kernel(hidden_states, w_router, w_gate, w_up, w_down):
    b, s, hd = hidden_states.shape
    t = b * s
    xf = hidden_states.reshape(t, hd)

    combine = pl.pallas_call(
        _router_body,
        out_shape=jax.ShapeDtypeStruct((t, NUM_EXPERTS), jnp.float32),
    )(xf, w_router)

    tm = 512
    nj = t // tm
    out = pl.pallas_call(
        _ffn_body,
        grid=(NUM_EXPERTS, nj),
        in_specs=[
            pl.BlockSpec((tm, hd), lambda e, j: (j, 0)),
            pl.BlockSpec((1, hd, INTER), lambda e, j: (e, 0, 0)),
            pl.BlockSpec((1, hd, INTER), lambda e, j: (e, 0, 0)),
            pl.BlockSpec((1, INTER, hd), lambda e, j: (e, 0, 0)),
            pl.BlockSpec((tm, NUM_EXPERTS), lambda e, j: (j, 0)),
        ],
        out_specs=pl.BlockSpec((t, hd), lambda e, j: (0, 0)),
        out_shape=jax.ShapeDtypeStruct((t, hd), jnp.float32),
        scratch_shapes=[pltpu.VMEM((t, hd), jnp.float32)],
    )(xf, w_gate, w_up, w_down, combine)

    return out.reshape(b, s, hd)


# speedup vs baseline: 1.6490x; 1.6490x over previous
import functools

import jax
import jax.numpy as jnp
from jax.experimental import pallas as pl
from jax.experimental.pallas import tpu as pltpu

NUM_EXPERTS = 8
TOP_K = 2
HIDDEN = 768
INTER = 2048


def _router_body(x_ref, wr_ref, comb_ref):
    x = x_ref[...]
    wr = wr_ref[...]
    logits = jax.lax.dot_general(
        x, wr, (((1,), (0,)), ((), ())), preferred_element_type=jnp.float32
    )
    aff = jax.nn.sigmoid(logits)
    t = aff.shape[0]
    iota = jax.lax.broadcasted_iota(jnp.int32, (t, NUM_EXPERTS), 1)
    m1 = jnp.max(aff, axis=-1, keepdims=True)
    im1 = jnp.min(jnp.where(aff == m1, iota, NUM_EXPERTS), axis=-1, keepdims=True)
    aff2 = jnp.where(iota == im1, -jnp.inf, aff)
    m2 = jnp.max(aff2, axis=-1, keepdims=True)
    im2 = jnp.min(jnp.where(aff2 == m2, iota, NUM_EXPERTS), axis=-1, keepdims=True)
    s = m1 + m2
    comb_ref[...] = (
        jnp.where(iota == im1, m1 / s, 0.0) + jnp.where(iota == im2, m2 / s, 0.0)
    ).astype(jnp.float32)


def _ffn_body(x_ref, wg_ref, wu_ref, wd_ref, comb_ref, out_ref, acc_ref):
    e = pl.program_id(0)
    j = pl.program_id(1)
    nj = pl.num_programs(1)

    x = x_ref[...]
    g = jax.lax.dot_general(
        x, wg_ref[0], (((1,), (0,)), ((), ())), preferred_element_type=jnp.float32
    )
    u = jax.lax.dot_general(
        x, wu_ref[0], (((1,), (0,)), ((), ())), preferred_element_type=jnp.float32
    )
    h = (g * jax.nn.sigmoid(g)) * u
    y = jax.lax.dot_general(
        h, wd_ref[0], (((1,), (0,)), ((), ())), preferred_element_type=jnp.float32
    )
    comb = comb_ref[...]
    eiota = jax.lax.broadcasted_iota(jnp.int32, comb.shape, 1)
    cw = jnp.sum(jnp.where(eiota == e, comb, 0.0), axis=-1, keepdims=True)
    y = y * cw

    tm = y.shape[0]
    row = pl.multiple_of(j * tm, tm)

    @pl.when(e == 0)
    def _init():
        acc_ref[pl.ds(row, tm), :] = y

    @pl.when(e > 0)
    def _acc():
        acc_ref[pl.ds(row, tm), :] = acc_ref[pl.ds(row, tm), :] + y

    @pl.when((e == pl.num_programs(0) - 1) & (j == nj - 1))
    def _fin():
        out_ref[...] = acc_ref[...]


@jax.jit
def kernel(hidden_states, w_router, w_gate, w_up, w_down):
    b, s, hd = hidden_states.shape
    t = b * s
    xf = hidden_states.reshape(t, hd)

    combine = pl.pallas_call(
        _router_body,
        out_shape=jax.ShapeDtypeStruct((t, NUM_EXPERTS), jnp.float32),
    )(xf, w_router)

    tm = 512
    nj = t // tm
    out = pl.pallas_call(
        _ffn_body,
        grid=(NUM_EXPERTS, nj),
        in_specs=[
            pl.BlockSpec((tm, hd), lambda e, j: (j, 0)),
            pl.BlockSpec((1, hd, INTER), lambda e, j: (e, 0, 0)),
            pl.BlockSpec((1, hd, INTER), lambda e, j: (e, 0, 0)),
            pl.BlockSpec((1, INTER, hd), lambda e, j: (e, 0, 0)),
            pl.BlockSpec((tm, NUM_EXPERTS), lambda e, j: (j, 0)),
        ],
        out_specs=pl.BlockSpec((t, hd), lambda e, j: (0, 0)),
        out_shape=jax.ShapeDtypeStruct((t, hd), jnp.float32),
        scratch_shapes=[pltpu.VMEM((t, hd), jnp.float32)],
    )(xf, w_gate, w_up, w_down, combine)

    return out.reshape(b, s, hd)
